# baseline (device time: 147705 ns/iter reference)
import jax
import jax.numpy as jnp
from jax import lax
from jax.experimental import pallas as pl
from jax.experimental.pallas import tpu as pltpu

N_DEV = 4


def kernel(x, w_mat, scale_x, scale_w):
    m_per, k = x.shape
    _, n_per = w_mat.shape
    half = m_per // 2

    x8 = x.astype(jnp.float8_e4m3fn)
    w8 = w_mat.astype(jnp.float8_e5m2)
    out_buf = jnp.zeros((N_DEV * m_per, n_per), jnp.float32)

    def body(x_ref, w_ref, sx_ref, sw_ref, out_alias_ref, out_hbm,
             recv_l, recv_r, recv_d, out_vmem,
             send_sems, recv_sems, copy_sems):
        my = lax.axis_index("i")
        left = lax.rem(my + (N_DEV - 1), N_DEV)
        right = lax.rem(my + 1, N_DEV)

        barrier_sem = pltpu.get_barrier_semaphore()
        pl.semaphore_signal(barrier_sem, inc=1, device_id=(left,),
                            device_id_type=pl.DeviceIdType.MESH)
        pl.semaphore_signal(barrier_sem, inc=1, device_id=(right,),
                            device_id_type=pl.DeviceIdType.MESH)
        pl.semaphore_wait(barrier_sem, 2)

        p1r = pltpu.make_async_remote_copy(
            src_ref=x_ref, dst_ref=recv_l,
            send_sem=send_sems.at[0], recv_sem=recv_sems.at[0],
            device_id=(right,), device_id_type=pl.DeviceIdType.MESH)
        p1l = pltpu.make_async_remote_copy(
            src_ref=x_ref, dst_ref=recv_r,
            send_sem=send_sems.at[1], recv_sem=recv_sems.at[1],
            device_id=(left,), device_id_type=pl.DeviceIdType.MESH)
        p1r.start()
        p1l.start()

        scale = sx_ref[0] * sw_ref[0]

        def block(a_ref, slot):
            acc = lax.dot_general(
                a_ref[...], w_ref[...],
                (((1,), (0,)), ((), ())),
                preferred_element_type=jnp.float32)
            out_vmem[slot] = jnp.maximum(acc * scale, 0.0)

        def store(slot, origin):
            cp = pltpu.make_async_copy(
                out_vmem.at[slot],
                out_hbm.at[pl.ds(origin * m_per, m_per)],
                copy_sems.at[slot])
            cp.start()
            return cp

        block(x_ref, 0)
        cp0 = store(0, my)

        p1r.wait_recv()
        p1l.wait_recv()

        p2r = pltpu.make_async_remote_copy(
            src_ref=recv_l.at[pl.ds(0, half)],
            dst_ref=recv_d.at[pl.ds(0, half)],
            send_sem=send_sems.at[2], recv_sem=recv_sems.at[2],
            device_id=(right,), device_id_type=pl.DeviceIdType.MESH)
        p2l = pltpu.make_async_remote_copy(
            src_ref=recv_r.at[pl.ds(half, half)],
            dst_ref=recv_d.at[pl.ds(half, half)],
            send_sem=send_sems.at[3], recv_sem=recv_sems.at[3],
            device_id=(left,), device_id_type=pl.DeviceIdType.MESH)
        p2r.start()
        p2l.start()

        block(recv_l, 1)
        cp1 = store(1, left)

        cp0.wait()
        block(recv_r, 0)
        cp2 = store(0, right)

        p2r.wait_recv()
        p2l.wait_recv()
        diag = lax.rem(my + 2, N_DEV)
        cp1.wait()
        block(recv_d, 1)
        cp3 = store(1, diag)

        cp2.wait()
        cp3.wait()
        p1r.wait_send()
        p1l.wait_send()
        p2r.wait_send()
        p2l.wait_send()

    return pl.pallas_call(
        body,
        out_shape=jax.ShapeDtypeStruct((N_DEV * m_per, n_per), jnp.float32),
        in_specs=[
            pl.BlockSpec(memory_space=pltpu.VMEM),
            pl.BlockSpec(memory_space=pltpu.VMEM),
            pl.BlockSpec(memory_space=pltpu.SMEM),
            pl.BlockSpec(memory_space=pltpu.SMEM),
            pl.BlockSpec(memory_space=pl.ANY),
        ],
        out_specs=pl.BlockSpec(memory_space=pl.ANY),
        scratch_shapes=[
            pltpu.VMEM((m_per, k), jnp.float8_e4m3fn),
            pltpu.VMEM((m_per, k), jnp.float8_e4m3fn),
            pltpu.VMEM((m_per, k), jnp.float8_e4m3fn),
            pltpu.VMEM((2, m_per, n_per), jnp.float32),
            pltpu.SemaphoreType.DMA((4,)),
            pltpu.SemaphoreType.DMA((4,)),
            pltpu.SemaphoreType.DMA((2,)),
        ],
        input_output_aliases={4: 0},
        compiler_params=pltpu.CompilerParams(
            collective_id=0,
            vmem_limit_bytes=60 * 1024 * 1024,
        ),
    )(x8, w8, scale_x, scale_w, out_buf)


# device time: 112830 ns/iter; 1.3091x vs baseline; 1.3091x over previous
import jax
import jax.numpy as jnp
from jax import lax
from jax.experimental import pallas as pl
from jax.experimental.pallas import tpu as pltpu

N_DEV = 4


def kernel(x, w_mat, scale_x, scale_w):
    m_per, k = x.shape
    _, n_per = w_mat.shape
    half = m_per // 2
    kt = k // 4

    def body(x_hbm, w_hbm, sx_ref, sw_ref, out_hbm,
             recv_l, recv_r, recv_d, out_vmem, x_f32, x8, w_f32, w8,
             send_sems, recv_sems, copy_sems, ld_sems):
        my = lax.axis_index("i")
        left = lax.rem(my + (N_DEV - 1), N_DEV)
        right = lax.rem(my + 1, N_DEV)

        barrier_sem = pltpu.get_barrier_semaphore()
        pl.semaphore_signal(barrier_sem, inc=1, device_id=(left,),
                            device_id_type=pl.DeviceIdType.MESH)
        pl.semaphore_signal(barrier_sem, inc=1, device_id=(right,),
                            device_id_type=pl.DeviceIdType.MESH)
        pl.semaphore_wait(barrier_sem, 2)

        ld_x0 = pltpu.make_async_copy(
            x_hbm.at[pl.ds(0, half)], x_f32, ld_sems.at[0])
        ld_x0.start()

        def p1(sem_idx, src, dst, dev):
            return pltpu.make_async_remote_copy(
                src_ref=src, dst_ref=dst,
                send_sem=send_sems.at[sem_idx],
                recv_sem=recv_sems.at[sem_idx],
                device_id=(dev,), device_id_type=pl.DeviceIdType.MESH)

        ld_x0.wait()
        ld_x1 = pltpu.make_async_copy(
            x_hbm.at[pl.ds(half, half)], x_f32, ld_sems.at[0])
        x8[pl.ds(0, half), :] = x_f32[...].astype(jnp.float8_e4m3fn)

        p1ru = p1(0, x8.at[pl.ds(0, half)], recv_l.at[pl.ds(0, half)], right)
        p1lu = p1(2, x8.at[pl.ds(0, half)], recv_r.at[pl.ds(0, half)], left)
        p1ru.start()
        p1lu.start()

        ld_x1.start()
        ld_x1.wait()
        x8[pl.ds(half, half), :] = x_f32[...].astype(jnp.float8_e4m3fn)

        p1rl = p1(1, x8.at[pl.ds(half, half)], recv_l.at[pl.ds(half, half)], right)
        p1ll = p1(3, x8.at[pl.ds(half, half)], recv_r.at[pl.ds(half, half)], left)
        p1rl.start()
        p1ll.start()

        for t in range(4):
            ld_w = pltpu.make_async_copy(
                w_hbm.at[pl.ds(t * kt, kt)], w_f32, ld_sems.at[1])
            ld_w.start()
            ld_w.wait()
            w8[pl.ds(t * kt, kt), :] = w_f32[...].astype(jnp.float8_e5m2)

        scale = sx_ref[0] * sw_ref[0]

        def block(a_ref, slot):
            acc = lax.dot_general(
                a_ref[...], w8[...],
                (((1,), (0,)), ((), ())),
                preferred_element_type=jnp.float32)
            out_vmem[slot] = jnp.maximum(acc * scale, 0.0)

        def store(slot, origin):
            cp = pltpu.make_async_copy(
                out_vmem.at[slot],
                out_hbm.at[pl.ds(origin * m_per, m_per)],
                copy_sems.at[slot])
            cp.start()
            return cp

        block(x8, 0)
        cp0 = store(0, my)

        p1ru.wait_recv()
        p2r = p1(4, recv_l.at[pl.ds(0, half)], recv_d.at[pl.ds(0, half)], right)
        p2r.start()
        p1ll.wait_recv()
        p2l = p1(5, recv_r.at[pl.ds(half, half)], recv_d.at[pl.ds(half, half)], left)
        p2l.start()

        p1rl.wait_recv()
        block(recv_l, 1)
        cp1 = store(1, left)

        p1lu.wait_recv()
        cp0.wait()
        block(recv_r, 0)
        cp2 = store(0, right)

        p2r.wait_recv()
        p2l.wait_recv()
        diag = lax.rem(my + 2, N_DEV)
        cp1.wait()
        block(recv_d, 1)
        cp3 = store(1, diag)

        cp2.wait()
        cp3.wait()
        p1ru.wait_send()
        p1rl.wait_send()
        p1lu.wait_send()
        p1ll.wait_send()
        p2r.wait_send()
        p2l.wait_send()

    return pl.pallas_call(
        body,
        out_shape=jax.ShapeDtypeStruct((N_DEV * m_per, n_per), jnp.float32),
        in_specs=[
            pl.BlockSpec(memory_space=pl.ANY),
            pl.BlockSpec(memory_space=pl.ANY),
            pl.BlockSpec(memory_space=pltpu.SMEM),
            pl.BlockSpec(memory_space=pltpu.SMEM),
        ],
        out_specs=pl.BlockSpec(memory_space=pl.ANY),
        scratch_shapes=[
            pltpu.VMEM((m_per, k), jnp.float8_e4m3fn),
            pltpu.VMEM((m_per, k), jnp.float8_e4m3fn),
            pltpu.VMEM((m_per, k), jnp.float8_e4m3fn),
            pltpu.VMEM((2, m_per, n_per), jnp.float32),
            pltpu.VMEM((half, k), jnp.float32),
            pltpu.VMEM((m_per, k), jnp.float8_e4m3fn),
            pltpu.VMEM((kt, n_per), jnp.float32),
            pltpu.VMEM((k, n_per), jnp.float8_e5m2),
            pltpu.SemaphoreType.DMA((6,)),
            pltpu.SemaphoreType.DMA((6,)),
            pltpu.SemaphoreType.DMA((2,)),
            pltpu.SemaphoreType.DMA((2,)),
        ],
        compiler_params=pltpu.CompilerParams(
            collective_id=0,
            vmem_limit_bytes=62 * 1024 * 1024,
        ),
    )(x, w_mat, scale_x, scale_w)


# device time: 104344 ns/iter; 1.4156x vs baseline; 1.0813x over previous
import jax
import jax.numpy as jnp
from jax import lax
from jax.experimental import pallas as pl
from jax.experimental.pallas import tpu as pltpu

N_DEV = 4


def kernel(x, w_mat, scale_x, scale_w):
    m_per, k = x.shape
    _, n_per = w_mat.shape
    half = m_per // 2
    quart = m_per // 4
    kt = k // 8

    def body(x_hbm, w_hbm, sx_ref, sw_ref, out_hbm,
             recv_l, recv_r, recv_d, out_vmem, x_f32, x8, w_f32, w8,
             send_sems, recv_sems, copy_sems, ld_sems):
        my = lax.axis_index("i")
        left = lax.rem(my + (N_DEV - 1), N_DEV)
        right = lax.rem(my + 1, N_DEV)
        diag = lax.rem(my + 2, N_DEV)

        barrier_sem = pltpu.get_barrier_semaphore()
        pl.semaphore_signal(barrier_sem, inc=1, device_id=(left,),
                            device_id_type=pl.DeviceIdType.MESH)
        pl.semaphore_signal(barrier_sem, inc=1, device_id=(right,),
                            device_id_type=pl.DeviceIdType.MESH)
        pl.semaphore_wait(barrier_sem, 2)

        def rdma(sem_idx, src, dst, dev):
            return pltpu.make_async_remote_copy(
                src_ref=src, dst_ref=dst,
                send_sem=send_sems.at[sem_idx],
                recv_sem=recv_sems.at[sem_idx],
                device_id=(dev,), device_id_type=pl.DeviceIdType.MESH)

        def ldx(q, slot):
            cp = pltpu.make_async_copy(
                x_hbm.at[pl.ds(q * quart, quart)], x_f32.at[slot],
                ld_sems.at[slot])
            cp.start()
            return cp

        lx = [ldx(0, 0), ldx(1, 1)]
        lx[0].wait()
        x8[pl.ds(0, quart), :] = x_f32[0].astype(jnp.float8_e4m3fn)
        lx[0] = ldx(2, 0)
        lx[1].wait()
        x8[pl.ds(quart, quart), :] = x_f32[1].astype(jnp.float8_e4m3fn)

        p1ru = rdma(0, x8.at[pl.ds(0, half)], recv_l.at[pl.ds(0, half)], right)
        p1lu = rdma(2, x8.at[pl.ds(0, half)], recv_r.at[pl.ds(0, half)], left)
        p1ru.start()
        p1lu.start()

        lx[1] = ldx(3, 1)
        lx[0].wait()
        x8[pl.ds(2 * quart, quart), :] = x_f32[0].astype(jnp.float8_e4m3fn)
        lx[1].wait()
        x8[pl.ds(3 * quart, quart), :] = x_f32[1].astype(jnp.float8_e4m3fn)

        p1rl = rdma(1, x8.at[pl.ds(half, half)], recv_l.at[pl.ds(half, half)], right)
        p1ll = rdma(3, x8.at[pl.ds(half, half)], recv_r.at[pl.ds(half, half)], left)
        p1rl.start()
        p1ll.start()

        def ldw(t, slot):
            cp = pltpu.make_async_copy(
                w_hbm.at[pl.ds(t * kt, kt)], w_f32.at[slot],
                ld_sems.at[2 + slot])
            cp.start()
            return cp

        lw = [ldw(0, 0), ldw(1, 1)]
        for t in range(8):
            slot = t % 2
            lw[slot].wait()
            w8[pl.ds(t * kt, kt), :] = (
                w_f32[slot].astype(jnp.float8_e5m2))
            if t + 2 < 8:
                lw[slot] = ldw(t + 2, slot)

        scale = sx_ref[0] * sw_ref[0]

        def gemm_rows(src_ref, src_off, rows, slot, slot_off, out_row):
            a = src_ref[pl.ds(src_off, rows), :]
            acc = lax.dot_general(
                a, w8[...], (((1,), (0,)), ((), ())),
                preferred_element_type=jnp.float32)
            out_vmem[slot, pl.ds(slot_off, rows), :] = (
                jnp.maximum(acc * scale, 0.0))

        def store_rows(sem_idx, rows, slot, slot_off, out_row):
            cp = pltpu.make_async_copy(
                out_vmem.at[slot, pl.ds(slot_off, rows)],
                out_hbm.at[pl.ds(out_row, rows)],
                copy_sems.at[sem_idx])
            cp.start()
            return cp

        def half_block(src_ref, src_off, slot, out_row, sem_idx):
            gemm_rows(src_ref, src_off, half, slot, 0, out_row)
            return store_rows(sem_idx, half, slot, 0, out_row)

        c0 = half_block(x8, 0, 0, my * m_per, 0)
        c1 = half_block(x8, half, 1, my * m_per + half, 1)

        p1ru.wait_recv()
        p2rq = [
            rdma(4, recv_l.at[pl.ds(0, quart)],
                 recv_d.at[pl.ds(0, quart)], right),
            rdma(5, recv_l.at[pl.ds(quart, quart)],
                 recv_d.at[pl.ds(quart, quart)], right),
        ]
        p2rq[0].start()
        p2rq[1].start()
        c2 = half_block(recv_l, 0, 2, left * m_per, 2)

        p1ll.wait_recv()
        p2lq = [
            rdma(6, recv_r.at[pl.ds(half, quart)],
                 recv_d.at[pl.ds(half, quart)], left),
            rdma(7, recv_r.at[pl.ds(3 * quart, quart)],
                 recv_d.at[pl.ds(3 * quart, quart)], left),
        ]
        p2lq[0].start()
        p2lq[1].start()
        c3 = half_block(recv_r, half, 3, right * m_per + half, 3)

        p1rl.wait_recv()
        c0.wait()
        c4 = half_block(recv_l, half, 0, left * m_per + half, 4)
        p1lu.wait_recv()
        c1.wait()
        c5 = half_block(recv_r, 0, 1, right * m_per, 5)

        p2rq[0].wait_recv()
        c2.wait()
        gemm_rows(recv_d, 0, quart, 2, 0, diag * m_per)
        c6 = store_rows(6, quart, 2, 0, diag * m_per)
        p2lq[0].wait_recv()
        c3.wait()
        gemm_rows(recv_d, half, quart, 3, 0, diag * m_per + half)
        c7 = store_rows(7, quart, 3, 0, diag * m_per + half)
        p2rq[1].wait_recv()
        gemm_rows(recv_d, quart, quart, 2, quart, diag * m_per + quart)
        c8 = store_rows(8, quart, 2, quart, diag * m_per + quart)
        p2lq[1].wait_recv()
        gemm_rows(recv_d, 3 * quart, quart, 3, quart,
                  diag * m_per + 3 * quart)
        c9 = store_rows(9, quart, 3, quart, diag * m_per + 3 * quart)

        c4.wait()
        c5.wait()
        c6.wait()
        c7.wait()
        c8.wait()
        c9.wait()
        p1ru.wait_send()
        p1rl.wait_send()
        p1lu.wait_send()
        p1ll.wait_send()
        p2rq[0].wait_send()
        p2rq[1].wait_send()
        p2lq[0].wait_send()
        p2lq[1].wait_send()

    return pl.pallas_call(
        body,
        out_shape=jax.ShapeDtypeStruct((N_DEV * m_per, n_per), jnp.float32),
        in_specs=[
            pl.BlockSpec(memory_space=pl.ANY),
            pl.BlockSpec(memory_space=pl.ANY),
            pl.BlockSpec(memory_space=pltpu.SMEM),
            pl.BlockSpec(memory_space=pltpu.SMEM),
        ],
        out_specs=pl.BlockSpec(memory_space=pl.ANY),
        scratch_shapes=[
            pltpu.VMEM((m_per, k), jnp.float8_e4m3fn),
            pltpu.VMEM((m_per, k), jnp.float8_e4m3fn),
            pltpu.VMEM((m_per, k), jnp.float8_e4m3fn),
            pltpu.VMEM((4, half, n_per), jnp.float32),
            pltpu.VMEM((2, quart, k), jnp.float32),
            pltpu.VMEM((m_per, k), jnp.float8_e4m3fn),
            pltpu.VMEM((2, kt, n_per), jnp.float32),
            pltpu.VMEM((k, n_per), jnp.float8_e5m2),
            pltpu.SemaphoreType.DMA((8,)),
            pltpu.SemaphoreType.DMA((8,)),
            pltpu.SemaphoreType.DMA((10,)),
            pltpu.SemaphoreType.DMA((4,)),
        ],
        compiler_params=pltpu.CompilerParams(
            collective_id=0,
            vmem_limit_bytes=62 * 1024 * 1024,
        ),
    )(x, w_mat, scale_x, scale_w)
